# baseline (speedup 1.0000x reference)
import functools

import jax
import jax.numpy as jnp
from jax import lax
from jax.experimental import pallas as pl
from jax.experimental.pallas import tpu as pltpu
from jax.experimental.pallas import tpu_sc as plsc

N = 1024
M = 1024
NB = 4
L = 16


def _chamfer_sc(x1p, x2t):
    mesh = plsc.VectorSubcoreMesh(core_axis_name="c", subcore_axis_name="s")

    @functools.partial(
        pl.kernel,
        mesh=mesh,
        out_type=jax.ShapeDtypeStruct((32, L), jnp.float32),
        scratch_types=[
            pltpu.VMEM((N // NB, L), jnp.float32),
            pltpu.VMEM((3, M), jnp.float32),
            pltpu.VMEM((M,), jnp.float32),
            pltpu.VMEM((M,), jnp.float32),
            pltpu.VMEM((L,), jnp.float32),
        ],
    )
    def k(x1_hbm, x2_hbm, out_hbm, a, b, sq2v, d2v, ov):
        wid = lax.axis_index("s") * 2 + lax.axis_index("c")
        p1 = wid // 4
        p2 = (wid // 16) * 4 + lax.rem(wid, 4)
        pltpu.sync_copy(x1_hbm.at[p1], a)
        pltpu.sync_copy(x2_hbm.at[p2], b)

        inf = jnp.float32(3.0e38)
        perms = [jnp.arange(L, dtype=jnp.int32) ^ (1 << k) for k in range(4)]
        dnums = lax.GatherDimensionNumbers(
            offset_dims=(), collapsed_slice_dims=(0,), start_index_map=(0,))

        def shuf(v, p):
            return lax.gather(
                v, p[:, None], dimension_numbers=dnums, slice_sizes=(1,),
                mode=lax.GatherScatterMode.PROMISE_IN_BOUNDS)

        def tree_min(v):
            for p in perms:
                v = jnp.minimum(v, shuf(v, p))
            return v

        def tree_sum(v):
            for p in perms:
                v = v + shuf(v, p)
            return v

        def init_body(g, _):
            s = pl.ds(g * L, L)
            bx = b[0, s]
            by = b[1, s]
            bz = b[2, s]
            sq2v[s] = bx * bx + by * by + bz * bz
            d2v[s] = jnp.full((L,), inf, jnp.float32)
            return 0

        lax.fori_loop(0, M // L, init_body, 0)

        def n_body(t, d1sum):
            cv = a[t]
            xs = [cv[u] for u in range(NB)]
            ys = [cv[NB + u] for u in range(NB)]
            zs = [cv[2 * NB + u] for u in range(NB)]
            sq1s = [xs[u] * xs[u] + ys[u] * ys[u] + zs[u] * zs[u]
                    for u in range(NB)]
            cxv = [jnp.full((L,), -2.0 * xs[u], jnp.float32) for u in range(NB)]
            cyv = [jnp.full((L,), -2.0 * ys[u], jnp.float32) for u in range(NB)]
            czv = [jnp.full((L,), -2.0 * zs[u], jnp.float32) for u in range(NB)]
            sq1v = [jnp.full((L,), sq1s[u], jnp.float32) for u in range(NB)]
            rmins = [jnp.full((L,), inf, jnp.float32) for _ in range(NB)]
            for mb in range(M // L):
                s = pl.ds(mb * L, L)
                bx = b[0, s]
                by = b[1, s]
                bz = b[2, s]
                s2 = sq2v[s]
                d2 = d2v[s]
                for u in range(NB):
                    tt = s2 + cxv[u] * bx + cyv[u] * by + czv[u] * bz
                    rmins[u] = jnp.minimum(rmins[u], tt)
                    d2 = jnp.minimum(d2, tt + sq1v[u])
                d2v[s] = d2
            for u in range(NB):
                d1sum = d1sum + sq1v[u] + tree_min(rmins[u])
            return d1sum

        d1sum = lax.fori_loop(0, N // NB, n_body,
                              jnp.zeros((L,), jnp.float32))

        def sum_body(g, acc):
            return acc + d2v[pl.ds(g * L, L)]

        d2part = lax.fori_loop(0, M // L, sum_body,
                               jnp.zeros((L,), jnp.float32))
        d2sum = tree_sum(d2part)

        res = d1sum * jnp.float32(1.0 / N) + d2sum * jnp.float32(1.0 / M)
        ov[:] = res
        pltpu.sync_copy(ov, out_hbm.at[wid])

    return k(x1p, x2t)


def kernel(xyz1_matrix, xyz2_matrix):
    B, S1, n, _ = xyz1_matrix.shape
    _, S2, m, _ = xyz2_matrix.shape
    x1g = xyz1_matrix.reshape(B * S1, n // NB, NB, 3).transpose(0, 1, 3, 2)
    x1p = jnp.concatenate(
        [x1g, jnp.zeros((B * S1, n // NB, 1, NB), jnp.float32)], axis=2
    ).reshape(B * S1, n // NB, L)
    x2t = xyz2_matrix.reshape(B * S2, m, 3).transpose(0, 2, 1)
    out = _chamfer_sc(x1p, x2t)
    return out[:, 0].reshape(B, S1, S2)


# speedup vs baseline: 2.3316x; 2.3316x over previous
import functools

import jax
import jax.numpy as jnp
from jax import lax
from jax.experimental import pallas as pl
from jax.experimental.pallas import tpu as pltpu
from jax.experimental.pallas import tpu_sc as plsc

N = 1024
M = 1024
NB = 4
L = 16


def _chamfer_sc(x1p, x2t):
    mesh = plsc.VectorSubcoreMesh(core_axis_name="c", subcore_axis_name="s")

    @functools.partial(
        pl.kernel,
        mesh=mesh,
        out_type=jax.ShapeDtypeStruct((32, L), jnp.float32),
        scratch_types=[
            pltpu.VMEM((N // NB, L), jnp.float32),
            pltpu.VMEM((3, M), jnp.float32),
            pltpu.VMEM((M,), jnp.float32),
            pltpu.VMEM((M,), jnp.float32),
            pltpu.VMEM((L,), jnp.float32),
        ],
    )
    def k(x1_hbm, x2_hbm, out_hbm, a, b, sq2v, d2v, ov):
        wid = lax.axis_index("s") * 2 + lax.axis_index("c")
        p1 = wid // 4
        p2 = (wid // 16) * 4 + lax.rem(wid, 4)
        pltpu.sync_copy(x1_hbm.at[p1], a)
        pltpu.sync_copy(x2_hbm.at[p2], b)

        inf = jnp.float32(3.0e38)
        perms = [jnp.arange(L, dtype=jnp.int32) ^ (1 << k) for k in range(4)]
        dnums = lax.GatherDimensionNumbers(
            offset_dims=(), collapsed_slice_dims=(0,), start_index_map=(0,))

        def shuf(v, p):
            return lax.gather(
                v, p[:, None], dimension_numbers=dnums, slice_sizes=(1,),
                mode=lax.GatherScatterMode.PROMISE_IN_BOUNDS)

        def tree_min(v):
            for p in perms:
                v = jnp.minimum(v, shuf(v, p))
            return v

        def tree_sum(v):
            for p in perms:
                v = v + shuf(v, p)
            return v

        def init_body(g, _):
            s = pl.ds(g * L, L)
            bx = b[0, s]
            by = b[1, s]
            bz = b[2, s]
            sq2v[s] = bx * bx + by * by + bz * bz
            d2v[s] = jnp.full((L,), inf, jnp.float32)
            return 0

        lax.fori_loop(0, M // L, init_body, 0)

        def n_body(t, d1sum):
            cv = a[t]
            xs = [cv[u] for u in range(NB)]
            ys = [cv[NB + u] for u in range(NB)]
            zs = [cv[2 * NB + u] for u in range(NB)]
            sq1s = [xs[u] * xs[u] + ys[u] * ys[u] + zs[u] * zs[u]
                    for u in range(NB)]
            cxv = [jnp.full((L,), -2.0 * xs[u], jnp.float32) for u in range(NB)]
            cyv = [jnp.full((L,), -2.0 * ys[u], jnp.float32) for u in range(NB)]
            czv = [jnp.full((L,), -2.0 * zs[u], jnp.float32) for u in range(NB)]
            sq1v = [jnp.full((L,), sq1s[u], jnp.float32) for u in range(NB)]
            rmins = [jnp.full((L,), inf, jnp.float32) for _ in range(NB)]
            for mb in range(M // L):
                s = pl.ds(mb * L, L)
                bx = b[0, s]
                by = b[1, s]
                bz = b[2, s]
                s2 = sq2v[s]
                d2 = d2v[s]
                for u in range(NB):
                    tt = s2 + cxv[u] * bx + cyv[u] * by + czv[u] * bz
                    rmins[u] = jnp.minimum(rmins[u], tt)
                    d2 = jnp.minimum(d2, tt + sq1v[u])
                d2v[s] = d2
            for u in range(NB):
                d1sum = d1sum + sq1v[u] + tree_min(rmins[u])
            return d1sum

        d1sum = lax.fori_loop(0, N // NB, n_body,
                              jnp.zeros((L,), jnp.float32))

        def sum_body(g, acc):
            return acc + d2v[pl.ds(g * L, L)]

        d2part = lax.fori_loop(0, M // L, sum_body,
                               jnp.zeros((L,), jnp.float32))
        d2sum = tree_sum(d2part)

        res = d1sum * jnp.float32(1.0 / N) + d2sum * jnp.float32(1.0 / M)
        ov[:] = res
        pltpu.sync_copy(ov, out_hbm.at[wid])

    return k(x1p, x2t)


def _tc_body(x1_ref, x2_ref, o_ref):
    a = x1_ref[0]
    nchunk = 4
    mc = M // nchunk
    rm128 = None
    cms = []
    for c in range(nchunk):
        dt = lax.dot_general(a, x2_ref[0, pl.ds(c * mc, mc), :],
                             (((1,), (1,)), ((), ())),
                             preferred_element_type=jnp.float32)
        h = dt
        while h.shape[1] > 128:
            half = h.shape[1] // 2
            h = jnp.minimum(h[:, :half], h[:, half:])
        rm128 = h if rm128 is None else jnp.minimum(rm128, h)
        cms.append(jnp.min(dt, axis=0))
    d1mean = jnp.mean(jnp.min(rm128, axis=1))
    d2mean = sum(jnp.mean(cm) for cm in cms) / nchunk
    o_ref[0] = jnp.full((8, 128), d1mean + d2mean, jnp.float32)


def _chamfer_tc(aug1, aug2, npairs):
    return pl.pallas_call(
        _tc_body,
        grid=(npairs,),
        in_specs=[
            pl.BlockSpec((1, N, 8), lambda p: (p // 4, 0, 0)),
            pl.BlockSpec((1, M, 8), lambda p: ((p // 16) * 4 + p % 4, 0, 0)),
        ],
        out_specs=pl.BlockSpec((1, 8, 128), lambda p: (p, 0, 0)),
        out_shape=jax.ShapeDtypeStruct((npairs, 8, 128), jnp.float32),
    )(aug1, aug2)


def _augment(x1, x2):
    sq1 = jnp.sum(x1 * x1, axis=-1, keepdims=True)
    sq2 = jnp.sum(x2 * x2, axis=-1, keepdims=True)
    one = jnp.ones_like(sq1)
    zero3 = jnp.zeros_like(x1)
    aug1 = jnp.concatenate([x1 * -2.0, sq1, one, zero3], axis=-1)
    aug2 = jnp.concatenate([x2, one, sq2, zero3], axis=-1)
    return aug1, aug2


def kernel(xyz1_matrix, xyz2_matrix):
    B, S1, n, _ = xyz1_matrix.shape
    _, S2, m, _ = xyz2_matrix.shape
    x1g = xyz1_matrix.reshape(B * S1, n // NB, NB, 3).transpose(0, 1, 3, 2)
    x1p = jnp.concatenate(
        [x1g, jnp.zeros((B * S1, n // NB, 1, NB), jnp.float32)], axis=2
    ).reshape(B * S1, n // NB, L)
    x2t = xyz2_matrix.reshape(B * S2, m, 3).transpose(0, 2, 1)
    aug1, aug2 = _augment(xyz1_matrix.reshape(B * S1, n, 3),
                          xyz2_matrix.reshape(B * S2, m, 3))
    out_tc = _chamfer_tc(aug1, aug2, 32)
    return out_tc[:, 0, 0].reshape(B, S1, S2)
